# baseline (speedup 1.0000x reference)
import jax
import jax.numpy as jnp
from jax import lax
from jax.experimental import pallas as pl
from jax.experimental.pallas import tpu as pltpu

H = 512
W = 512
STUFF = 53
THING = 80
NUM_INST = 64


def _body(sem_ref, out_ref, csem):
    n = pl.program_id(0)
    pltpu.make_async_copy(sem_ref.at[n], out_ref.at[n], csem).start()

    @pl.when(n == STUFF + NUM_INST - 1)
    def _():
        def wait_copy(i, carry):
            pltpu.make_async_copy(sem_ref.at[i], out_ref.at[i], csem).wait()
            return carry
        lax.fori_loop(0, STUFF + NUM_INST, wait_copy, 0)


def kernel(sem_seg_logits, mask_logits, boxes, cls_idx):
    sem = sem_seg_logits.reshape(STUFF + THING, H, W)
    out = pl.pallas_call(
        _body,
        grid=(STUFF + NUM_INST,),
        in_specs=[pl.BlockSpec(memory_space=pl.ANY)],
        out_specs=pl.BlockSpec(memory_space=pl.ANY),
        out_shape=jax.ShapeDtypeStruct((STUFF + NUM_INST, H, W), jnp.float32),
        scratch_shapes=[pltpu.SemaphoreType.DMA],
    )(sem)
    return out.reshape(1, STUFF + NUM_INST, H, W)


# speedup vs baseline: 42.1367x; 42.1367x over previous
import jax
import jax.numpy as jnp
from jax import lax
from jax.experimental import pallas as pl
from jax.experimental.pallas import tpu as pltpu

H = 512
W = 512
STUFF = 53
THING = 80
NUM_INST = 64
MSIZE = 100
BLK = 128
STRIP = 2 * BLK

_CH, _BY0, _BX0, _BH, _BW, _CY2, _CX2 = range(7)


def _body(s, sem_ref, mask_ref, out_ref, gbuf, sbuf, zbuf, cbuf,
          gsem, ssem, zsem, cfsem, cssem):
    n = pl.program_id(0)

    def hbase_of(k):
        return jnp.minimum(s[_BY0, k] // BLK, H // BLK - 2) * BLK

    def gfetch(k):
        pltpu.make_async_copy(
            sem_ref.at[s[_CH, k], pl.ds(hbase_of(k), STRIP), :],
            gbuf.at[k % 3], gsem.at[k % 3]).start()

    def cfetch(c):
        return pltpu.make_async_copy(sem_ref.at[c], cbuf.at[c % 3],
                                     cfsem.at[c % 3])

    def cstore(c):
        return pltpu.make_async_copy(cbuf.at[c % 3], out_ref.at[c],
                                     cssem.at[c % 3])

    @pl.when(n == 0)
    def _():
        zbuf[...] = jnp.zeros_like(zbuf)
        gfetch(0)
        gfetch(1)
        cfetch(0).start()
        cfetch(1).start()

    @pl.when(n + 2 < STUFF)
    def _():
        @pl.when(n >= 1)
        def _():
            cstore(n - 1).wait()

        cfetch(n + 2).start()

    @pl.when(n < STUFF)
    def _():
        cfetch(n).wait()
        cstore(n).start()

    by0 = s[_BY0, n]
    bx0 = s[_BX0, n]
    bh = s[_BH, n]
    bw = s[_BW, n]
    cy2 = s[_CY2, n]
    cx2 = s[_CX2, n]
    hbase = hbase_of(n)
    by0f = by0.astype(jnp.float32)
    bx0f = bx0.astype(jnp.float32)
    bhf = bh.astype(jnp.float32)
    bwf = bw.astype(jnp.float32)

    h = (lax.broadcasted_iota(jnp.int32, (STRIP, 128), 0) + hbase).astype(jnp.float32)
    m = lax.broadcasted_iota(jnp.int32, (STRIP, 128), 1).astype(jnp.float32)
    sy = (h - by0f + 0.5) * (MSIZE / bhf) - 0.5
    sy = jnp.clip(sy, 0.0, MSIZE - 1.0)
    yf = jnp.floor(sy)
    wy = sy - yf
    ay = (m == yf) * (1.0 - wy) + (m == jnp.minimum(yf + 1.0, MSIZE - 1.0)) * wy
    rowin = (h >= by0f) & (h <= by0f + bhf - 1.0)
    ay = jnp.where(rowin, ay, 0.0)

    k_ = lax.broadcasted_iota(jnp.int32, (128, W), 0).astype(jnp.float32)
    xx = lax.broadcasted_iota(jnp.int32, (128, W), 1).astype(jnp.float32)
    sx = (xx - bx0f + 0.5) * (MSIZE / bwf) - 0.5
    sx = jnp.clip(sx, 0.0, MSIZE - 1.0)
    xf = jnp.floor(sx)
    wx = sx - xf
    axt = (k_ == xf) * (1.0 - wx) + (k_ == jnp.minimum(xf + 1.0, MSIZE - 1.0)) * wx
    colin = (xx >= bx0f) & (xx <= bx0f + bwf - 1.0)
    axt = jnp.where(colin, axt, 0.0)

    t = jnp.dot(ay, mask_ref[0], precision=lax.Precision.DEFAULT,
                preferred_element_type=jnp.float32)
    p = jnp.dot(t, axt, precision=lax.Precision.DEFAULT,
                preferred_element_type=jnp.float32)

    pltpu.make_async_copy(
        sem_ref.at[s[_CH, n], pl.ds(hbase, STRIP), :],
        gbuf.at[n % 3], gsem.at[n % 3]).wait()
    hi = lax.broadcasted_iota(jnp.int32, (STRIP, W), 0) + hbase
    xi = lax.broadcasted_iota(jnp.int32, (STRIP, W), 1)
    cm = (hi >= by0) & (hi < cy2) & (xi >= bx0) & (xi < cx2)
    res = p + jnp.where(cm, gbuf[n % 3], 0.0)

    def strip_copy(k):
        return pltpu.make_async_copy(
            sbuf.at[k % 2], out_ref.at[STUFF + k, pl.ds(hbase_of(k), STRIP), :],
            ssem.at[k % 2])

    @pl.when(n >= 2)
    def _():
        strip_copy(n - 2).wait()

    sbuf[n % 2] = res
    strip_copy(n).start()

    jlo = by0 // BLK
    zoff_a = jnp.where(jlo == 0, 2 * BLK, 0)
    zoff_b = jnp.where(jlo >= 2, BLK, 3 * BLK)
    pltpu.make_async_copy(
        zbuf.at[pl.ds(0, BLK)], out_ref.at[STUFF + n, pl.ds(zoff_a, BLK), :],
        zsem).start()
    pltpu.make_async_copy(
        zbuf.at[pl.ds(0, BLK)], out_ref.at[STUFF + n, pl.ds(zoff_b, BLK), :],
        zsem).start()

    @pl.when(n + 2 < NUM_INST)
    def _():
        gfetch(n + 2)

    @pl.when(n == NUM_INST - 1)
    def _():
        cstore(STUFF - 3).wait()
        cstore(STUFF - 2).wait()
        cstore(STUFF - 1).wait()

        def wait_zero(i, carry):
            pltpu.make_async_copy(
                zbuf.at[pl.ds(0, BLK)], out_ref.at[STUFF, pl.ds(0, BLK), :],
                zsem).wait()
            return carry

        lax.fori_loop(0, 2 * NUM_INST, wait_zero, 0)
        strip_copy(n - 1).wait()
        strip_copy(n).wait()


def _grid_spec():
    return pltpu.PrefetchScalarGridSpec(
        num_scalar_prefetch=1,
        grid=(NUM_INST,),
        in_specs=[
            pl.BlockSpec(memory_space=pl.ANY),
            pl.BlockSpec((1, 128, 128), lambda n, s: (n, 0, 0)),
        ],
        out_specs=pl.BlockSpec(memory_space=pl.ANY),
        scratch_shapes=[
            pltpu.VMEM((3, STRIP, W), jnp.float32),
            pltpu.VMEM((2, STRIP, W), jnp.float32),
            pltpu.VMEM((BLK, W), jnp.float32),
            pltpu.VMEM((3, H, W), jnp.float32),
            pltpu.SemaphoreType.DMA((3,)),
            pltpu.SemaphoreType.DMA((2,)),
            pltpu.SemaphoreType.DMA,
            pltpu.SemaphoreType.DMA((3,)),
            pltpu.SemaphoreType.DMA((3,)),
        ],
    )


def _prep(sem_seg_logits, mask_logits, boxes, cls_idx):
    sem = sem_seg_logits.reshape(STUFF + THING, H, W)
    mask = mask_logits.reshape(NUM_INST, MSIZE, MSIZE)
    maskp = jnp.pad(mask, ((0, 0), (0, 128 - MSIZE), (0, 128 - MSIZE)))
    bx0 = boxes[:, 0].astype(jnp.int32)
    by0 = boxes[:, 1].astype(jnp.int32)
    bx1 = boxes[:, 2].astype(jnp.int32)
    by1 = boxes[:, 3].astype(jnp.int32)
    bw = bx1 - bx0 + 1
    bh = by1 - by0 + 1
    cx2 = jnp.round(boxes[:, 2]).astype(jnp.int32) + 1
    cy2 = jnp.round(boxes[:, 3]).astype(jnp.int32) + 1
    ch = STUFF + cls_idx.astype(jnp.int32)
    scal = jnp.stack([ch, by0, bx0, bh, bw, cy2, cx2,
                      jnp.zeros_like(ch)])
    return scal, sem, maskp


def kernel(sem_seg_logits, mask_logits, boxes, cls_idx):
    scal, sem, maskp = _prep(sem_seg_logits, mask_logits, boxes, cls_idx)
    out = pl.pallas_call(
        _body,
        grid_spec=_grid_spec(),
        out_shape=jax.ShapeDtypeStruct((STUFF + NUM_INST, H, W), jnp.float32),
        compiler_params=pltpu.CompilerParams(
            dimension_semantics=("arbitrary",)),
    )(scal, sem, maskp)
    return out.reshape(1, STUFF + NUM_INST, H, W)


# speedup vs baseline: 44.0375x; 1.0451x over previous
import jax
import jax.numpy as jnp
from jax import lax
from jax.experimental import pallas as pl
from jax.experimental.pallas import tpu as pltpu

H = 512
W = 512
STUFF = 53
THING = 80
NUM_INST = 64
MSIZE = 100
BLK = 128
STRIP = 2 * BLK

_CH, _BY0, _BX0, _BH, _BW, _CY2, _CX2 = range(7)


def _body(s, sem_ref, mask_ref, out_ref, gbuf, sbuf, zbuf, cbuf,
          gsem, ssem, zsem, wsem, cfsem, cssem):
    n = pl.program_id(0)

    def hbase_of(k):
        return jnp.minimum(s[_BY0, k] // BLK, H // BLK - 2) * BLK

    def wbase_of(k):
        return jnp.minimum(s[_BX0, k] // BLK, W // BLK - 2) * BLK

    def gfetch(k):
        pltpu.make_async_copy(
            sem_ref.at[s[_CH, k], pl.ds(hbase_of(k), STRIP),
                       pl.ds(wbase_of(k), STRIP)],
            gbuf.at[k % 3], gsem.at[k % 3]).start()

    def cfetch(c):
        return pltpu.make_async_copy(sem_ref.at[c], cbuf.at[c % 3],
                                     cfsem.at[c % 3])

    def cstore(c):
        return pltpu.make_async_copy(cbuf.at[c % 3], out_ref.at[c],
                                     cssem.at[c % 3])

    @pl.when(n == 0)
    def _():
        zbuf[...] = jnp.zeros_like(zbuf)
        gfetch(0)
        gfetch(1)
        cfetch(0).start()
        cfetch(1).start()

    @pl.when(n + 2 < STUFF)
    def _():
        @pl.when(n >= 1)
        def _():
            cstore(n - 1).wait()

        cfetch(n + 2).start()

    @pl.when(n < STUFF)
    def _():
        cfetch(n).wait()
        cstore(n).start()

    by0 = s[_BY0, n]
    bx0 = s[_BX0, n]
    bh = s[_BH, n]
    bw = s[_BW, n]
    cy2 = s[_CY2, n]
    cx2 = s[_CX2, n]
    hbase = hbase_of(n)
    by0f = by0.astype(jnp.float32)
    bx0f = bx0.astype(jnp.float32)
    bhf = bh.astype(jnp.float32)
    bwf = bw.astype(jnp.float32)

    h = (lax.broadcasted_iota(jnp.int32, (STRIP, 128), 0) + hbase).astype(jnp.float32)
    m = lax.broadcasted_iota(jnp.int32, (STRIP, 128), 1).astype(jnp.float32)
    sy = (h - by0f + 0.5) * (MSIZE / bhf) - 0.5
    sy = jnp.clip(sy, 0.0, MSIZE - 1.0)
    yf = jnp.floor(sy)
    wy = sy - yf
    ay = (m == yf) * (1.0 - wy) + (m == jnp.minimum(yf + 1.0, MSIZE - 1.0)) * wy
    rowin = (h >= by0f) & (h <= by0f + bhf - 1.0)
    ay = jnp.where(rowin, ay, 0.0)

    wbase = wbase_of(n)
    k_ = lax.broadcasted_iota(jnp.int32, (128, STRIP), 0).astype(jnp.float32)
    xx = (lax.broadcasted_iota(jnp.int32, (128, STRIP), 1) + wbase).astype(jnp.float32)
    sx = (xx - bx0f + 0.5) * (MSIZE / bwf) - 0.5
    sx = jnp.clip(sx, 0.0, MSIZE - 1.0)
    xf = jnp.floor(sx)
    wx = sx - xf
    axt = (k_ == xf) * (1.0 - wx) + (k_ == jnp.minimum(xf + 1.0, MSIZE - 1.0)) * wx
    colin = (xx >= bx0f) & (xx <= bx0f + bwf - 1.0)
    axt = jnp.where(colin, axt, 0.0)

    t = jnp.dot(ay, mask_ref[0], precision=lax.Precision.DEFAULT,
                preferred_element_type=jnp.float32)
    p = jnp.dot(t, axt, precision=lax.Precision.DEFAULT,
                preferred_element_type=jnp.float32)

    pltpu.make_async_copy(
        sem_ref.at[s[_CH, n], pl.ds(hbase, STRIP), pl.ds(wbase, STRIP)],
        gbuf.at[n % 3], gsem.at[n % 3]).wait()
    hi = lax.broadcasted_iota(jnp.int32, (STRIP, STRIP), 0) + hbase
    xi = lax.broadcasted_iota(jnp.int32, (STRIP, STRIP), 1) + wbase
    cm = (hi >= by0) & (hi < cy2) & (xi >= bx0) & (xi < cx2)
    res = p + jnp.where(cm, gbuf[n % 3], 0.0)

    def strip_copy(k):
        return pltpu.make_async_copy(
            sbuf.at[k % 2],
            out_ref.at[STUFF + k, pl.ds(hbase_of(k), STRIP),
                       pl.ds(wbase_of(k), STRIP)],
            ssem.at[k % 2])

    @pl.when(n >= 2)
    def _():
        strip_copy(n - 2).wait()

    sbuf[n % 2] = res
    strip_copy(n).start()

    jlo = by0 // BLK
    zoff_a = jnp.where(jlo == 0, 2 * BLK, 0)
    zoff_b = jnp.where(jlo >= 2, BLK, 3 * BLK)
    klo = bx0 // BLK
    woff_a = jnp.where(klo == 0, 2 * BLK, 0)
    woff_b = jnp.where(klo >= 2, BLK, 3 * BLK)
    pltpu.make_async_copy(
        zbuf.at[pl.ds(0, BLK), :], out_ref.at[STUFF + n, pl.ds(zoff_a, BLK), :],
        zsem).start()
    pltpu.make_async_copy(
        zbuf.at[pl.ds(0, BLK), :], out_ref.at[STUFF + n, pl.ds(zoff_b, BLK), :],
        zsem).start()
    pltpu.make_async_copy(
        zbuf.at[pl.ds(0, STRIP), pl.ds(0, BLK)],
        out_ref.at[STUFF + n, pl.ds(hbase, STRIP), pl.ds(woff_a, BLK)],
        wsem).start()
    pltpu.make_async_copy(
        zbuf.at[pl.ds(0, STRIP), pl.ds(0, BLK)],
        out_ref.at[STUFF + n, pl.ds(hbase, STRIP), pl.ds(woff_b, BLK)],
        wsem).start()

    @pl.when(n + 2 < NUM_INST)
    def _():
        gfetch(n + 2)

    @pl.when(n == NUM_INST - 1)
    def _():
        cstore(STUFF - 3).wait()
        cstore(STUFF - 2).wait()
        cstore(STUFF - 1).wait()

        def wait_zero_row(i, carry):
            pltpu.make_async_copy(
                zbuf.at[pl.ds(0, BLK), :], out_ref.at[STUFF, pl.ds(0, BLK), :],
                zsem).wait()
            return carry

        lax.fori_loop(0, 2 * NUM_INST, wait_zero_row, 0)

        def wait_zero_col(i, carry):
            pltpu.make_async_copy(
                zbuf.at[pl.ds(0, STRIP), pl.ds(0, BLK)],
                out_ref.at[STUFF, pl.ds(0, STRIP), pl.ds(0, BLK)],
                wsem).wait()
            return carry

        lax.fori_loop(0, 2 * NUM_INST, wait_zero_col, 0)
        strip_copy(n - 1).wait()
        strip_copy(n).wait()


def _grid_spec():
    return pltpu.PrefetchScalarGridSpec(
        num_scalar_prefetch=1,
        grid=(NUM_INST,),
        in_specs=[
            pl.BlockSpec(memory_space=pl.ANY),
            pl.BlockSpec((1, 128, 128), lambda n, s: (n, 0, 0)),
        ],
        out_specs=pl.BlockSpec(memory_space=pl.ANY),
        scratch_shapes=[
            pltpu.VMEM((3, STRIP, STRIP), jnp.float32),
            pltpu.VMEM((2, STRIP, STRIP), jnp.float32),
            pltpu.VMEM((STRIP, W), jnp.float32),
            pltpu.VMEM((3, H, W), jnp.float32),
            pltpu.SemaphoreType.DMA((3,)),
            pltpu.SemaphoreType.DMA((2,)),
            pltpu.SemaphoreType.DMA,
            pltpu.SemaphoreType.DMA,
            pltpu.SemaphoreType.DMA((3,)),
            pltpu.SemaphoreType.DMA((3,)),
        ],
    )


def _prep(sem_seg_logits, mask_logits, boxes, cls_idx):
    sem = sem_seg_logits.reshape(STUFF + THING, H, W)
    mask = mask_logits.reshape(NUM_INST, MSIZE, MSIZE)
    maskp = jnp.pad(mask, ((0, 0), (0, 128 - MSIZE), (0, 128 - MSIZE)))
    bx0 = boxes[:, 0].astype(jnp.int32)
    by0 = boxes[:, 1].astype(jnp.int32)
    bx1 = boxes[:, 2].astype(jnp.int32)
    by1 = boxes[:, 3].astype(jnp.int32)
    bw = bx1 - bx0 + 1
    bh = by1 - by0 + 1
    cx2 = jnp.round(boxes[:, 2]).astype(jnp.int32) + 1
    cy2 = jnp.round(boxes[:, 3]).astype(jnp.int32) + 1
    ch = STUFF + cls_idx.astype(jnp.int32)
    scal = jnp.stack([ch, by0, bx0, bh, bw, cy2, cx2,
                      jnp.zeros_like(ch)])
    return scal, sem, maskp


def kernel(sem_seg_logits, mask_logits, boxes, cls_idx):
    scal, sem, maskp = _prep(sem_seg_logits, mask_logits, boxes, cls_idx)
    out = pl.pallas_call(
        _body,
        grid_spec=_grid_spec(),
        out_shape=jax.ShapeDtypeStruct((STUFF + NUM_INST, H, W), jnp.float32),
        compiler_params=pltpu.CompilerParams(
            dimension_semantics=("arbitrary",)),
    )(scal, sem, maskp)
    return out.reshape(1, STUFF + NUM_INST, H, W)
